# baseline (speedup 1.0000x reference)
import functools
import math

import jax
import jax.numpy as jnp
from jax import lax
from jax.experimental import pallas as pl
from jax.experimental.pallas import tpu as pltpu
from jax.experimental.pallas import tpu_sc as plsc

_NC = 2
_NS = 16
_NW = _NC * _NS

_D = 768
_GROUPS = _D // 16
_KP = 16
_RING = 2


def _body(ids_hbm, table_hbm, pe_hbm, out_hbm,
          idx_v, pe_v, rows_v, sem_idx, sems_pe, sems_g, sems_s):
    batch, seq_len = ids_hbm.shape
    pos_per_w = seq_len // _NW
    n_tasks = pos_per_w // _KP
    scale = jnp.float32(math.sqrt(float(_D)))

    w = lax.axis_index("s") * _NC + lax.axis_index("c")
    w_base = w * pos_per_w

    def issue_pe(t):
        p = t % _RING
        return pltpu.async_copy(
            pe_hbm.at[0, pl.ds(w_base + t * _KP, _KP), :],
            pe_v.at[p], sems_pe[p])

    def issue_gather(t):
        p = t % _RING
        return pltpu.async_copy(
            table_hbm.at[idx_v.at[t]], rows_v.at[p], sems_g[p])

    def issue_stores(t):
        p = t % _RING
        return [
            pltpu.async_copy(
                rows_v.at[p, pl.ds(b * _KP, _KP), :],
                out_hbm.at[b, pl.ds(w_base + t * _KP, _KP), :], sems_s[p])
            for b in range(batch)
        ]

    def compute(t):
        p = t % _RING

        def row_fma(i, carry):
            for b in range(batch):
                r = b * _KP
                for j in range(_GROUPS):
                    sl = pl.ds(j * 16, 16)
                    rows_v[p, r + i, sl] = (rows_v[p, r + i, sl] * scale
                                            + pe_v[p, i, sl])
            return carry
        lax.fori_loop(0, _KP, row_fma, 0)

    idx_cps = [
        pltpu.async_copy(
            ids_hbm.at[b, pl.ds(w_base + t * _KP, _KP)],
            idx_v.at[t, pl.ds(b * _KP, _KP)], sem_idx)
        for t in range(n_tasks) for b in range(batch)
    ]
    for cp in idx_cps:
        cp.wait()

    pes = {0: issue_pe(0), 1: issue_pe(1)}
    gathers = {0: issue_gather(0), 1: issue_gather(1)}
    stores = {}

    for t in range(n_tasks):
        pes[t].wait()
        gathers[t].wait()
        compute(t)
        stores[t] = issue_stores(t)
        if t + _RING < n_tasks:
            pes[t + _RING] = issue_pe(t + _RING)
            for cp in stores[t]:
                cp.wait()
            gathers[t + _RING] = issue_gather(t + _RING)
    for t in range(max(0, n_tasks - _RING), n_tasks):
        for cp in stores[t]:
            cp.wait()


def kernel(input_ids, word_embeddings, pe):
    batch, seq_len = input_ids.shape
    ids32 = input_ids.astype(jnp.int32)
    pos_per_w = seq_len // _NW
    n_tasks = pos_per_w // _KP

    mesh = plsc.VectorSubcoreMesh(
        core_axis_name="c", subcore_axis_name="s",
        num_cores=_NC, num_subcores=_NS,
    )
    run = pl.kernel(
        _body,
        out_type=jax.ShapeDtypeStruct((batch, seq_len, _D), jnp.float32),
        mesh=mesh,
        scratch_types=[
            pltpu.VMEM((n_tasks, batch * _KP), jnp.int32),
            pltpu.VMEM((_RING, _KP, _D), jnp.float32),
            pltpu.VMEM((_RING, batch * _KP, _D), jnp.float32),
            pltpu.SemaphoreType.DMA,
            [pltpu.SemaphoreType.DMA] * _RING,
            [pltpu.SemaphoreType.DMA] * _RING,
            [pltpu.SemaphoreType.DMA] * _RING,
        ],
    )
    return run(ids32, word_embeddings, pe)


# speedup vs baseline: 1.0273x; 1.0273x over previous
import functools
import math

import jax
import jax.numpy as jnp
from jax import lax
from jax.experimental import pallas as pl
from jax.experimental.pallas import tpu as pltpu
from jax.experimental.pallas import tpu_sc as plsc

_NC = 2
_NS = 16
_NW = _NC * _NS

_D = 768
_GROUPS = _D // 16
_K = 32
_RING = 3


def _body(ids_hbm, table_hbm, pe_hbm, out_hbm, idx_v, pe_v, rows_v,
          sg0, sg1, sg2, ss0, ss1, ss2, sp0, sp1):
    batch, seq_len = ids_hbm.shape
    pos_per_w = seq_len // _NW
    n_chunks = pos_per_w // _K
    n_tasks = n_chunks * batch
    scale = jnp.float32(math.sqrt(float(_D)))

    sems_g = [sg0, sg1, sg2]
    sems_s = [ss0, ss1, ss2]
    sems_p = [sp0, sp1]

    w = lax.axis_index("s") * _NC + lax.axis_index("c")
    w_base = w * pos_per_w

    def issue_pe(pc):
        return pltpu.async_copy(
            pe_hbm.at[0, pl.ds(w_base + pc * _K, _K), :],
            pe_v.at[pc % 2], sems_p[pc % 2])

    def issue_gather(t):
        pc, b = divmod(t, batch)
        return pltpu.async_copy(
            table_hbm.at[idx_v.at[b, pl.ds(pc * _K, _K)]],
            rows_v.at[t % _RING], sems_g[t % _RING])

    def issue_store(t):
        pc, b = divmod(t, batch)
        return pltpu.async_copy(
            rows_v.at[t % _RING],
            out_hbm.at[b, pl.ds(w_base + pc * _K, _K), :],
            sems_s[t % _RING])

    def compute(t):
        pc = t // batch
        p = t % _RING
        pb = pc % 2

        def row_fma(i, carry):
            for j in range(_GROUPS):
                sl = pl.ds(j * 16, 16)
                rows_v[p, i, sl] = rows_v[p, i, sl] * scale + pe_v[pb, i, sl]
            return carry
        lax.fori_loop(0, _K, row_fma, 0)

    for b in range(batch):
        pltpu.sync_copy(ids_hbm.at[b, pl.ds(w_base, pos_per_w)],
                        idx_v.at[b])

    pes = {0: issue_pe(0)}
    gathers = {0: issue_gather(0)}
    stores = {}

    pes[0].wait()
    for t in range(n_tasks):
        if t + 1 < n_tasks:
            if t - 2 >= 0:
                stores[t - 2].wait()
            gathers[t + 1] = issue_gather(t + 1)
        if t == 2:
            pes[1] = issue_pe(1)
        if t == batch:
            pes[1].wait()
        gathers[t].wait()
        compute(t)
        stores[t] = issue_store(t)
    for t in range(n_tasks - _RING, n_tasks):
        stores[t].wait()


def kernel(input_ids, word_embeddings, pe):
    batch, seq_len = input_ids.shape
    ids32 = input_ids.astype(jnp.int32)
    pos_per_w = seq_len // _NW

    mesh = plsc.VectorSubcoreMesh(
        core_axis_name="c", subcore_axis_name="s",
        num_cores=_NC, num_subcores=_NS,
    )
    run = pl.kernel(
        _body,
        out_type=jax.ShapeDtypeStruct((batch, seq_len, _D), jnp.float32),
        mesh=mesh,
        scratch_types=[
            pltpu.VMEM((batch, pos_per_w), jnp.int32),
            pltpu.VMEM((2, _K, _D), jnp.float32),
            pltpu.VMEM((_RING, _K, _D), jnp.float32),
            pltpu.SemaphoreType.DMA,
            pltpu.SemaphoreType.DMA,
            pltpu.SemaphoreType.DMA,
            pltpu.SemaphoreType.DMA,
            pltpu.SemaphoreType.DMA,
            pltpu.SemaphoreType.DMA,
            pltpu.SemaphoreType.DMA,
            pltpu.SemaphoreType.DMA,
        ],
    )
    return run(ids32, word_embeddings, pe)


# speedup vs baseline: 1.5677x; 1.5261x over previous
import functools
import math

import jax
import jax.numpy as jnp
from jax import lax
from jax.experimental import pallas as pl
from jax.experimental.pallas import tpu as pltpu
from jax.experimental.pallas import tpu_sc as plsc

_NC = 2
_NS = 16
_NW = _NC * _NS

_D = 768
_GROUPS = _D // 16
_K = 32
_RING = 3


def _body(ids_hbm, table_hbm, pe_hbm, out_hbm, idx_v, pe0, pe1,
          rows0, rows1, rows2, sg0, sg1, sg2, ss0, ss1, ss2, sp0, sp1):
    batch, seq_len = ids_hbm.shape
    pos_per_w = seq_len // _NW
    n_chunks = pos_per_w // _K
    n_tasks = n_chunks * batch
    scale = jnp.float32(math.sqrt(float(_D)))

    rows = [rows0, rows1, rows2]
    pes_v = [pe0, pe1]
    sems_g = [sg0, sg1, sg2]
    sems_s = [ss0, ss1, ss2]
    sems_p = [sp0, sp1]

    w = lax.axis_index("s") * _NC + lax.axis_index("c")
    w_base = w * pos_per_w

    def issue_pe(pc):
        return pltpu.async_copy(
            pe_hbm.at[0, pl.ds(w_base + pc * _K, _K), :],
            pes_v[pc % 2], sems_p[pc % 2])

    def issue_gather(t):
        pc, b = divmod(t, batch)
        return pltpu.async_copy(
            table_hbm.at[idx_v.at[b, pl.ds(pc * _K, _K)]],
            rows[t % _RING], sems_g[t % _RING])

    def issue_store(t):
        pc, b = divmod(t, batch)
        return pltpu.async_copy(
            rows[t % _RING],
            out_hbm.at[b, pl.ds(w_base + pc * _K, _K), :],
            sems_s[t % _RING])

    def compute(t):
        pc = t // batch
        rv = rows[t % _RING]
        pv = pes_v[pc % 2]

        def row_fma(i, carry):
            for j in range(_GROUPS):
                sl = pl.ds(j * 16, 16)
                rv[i, sl] = rv[i, sl] * scale + pv[i, sl]
            return carry
        lax.fori_loop(0, _K, row_fma, 0)

    for b in range(batch):
        pltpu.sync_copy(ids_hbm.at[b, pl.ds(w_base, pos_per_w)],
                        idx_v.at[b])

    pes = {0: issue_pe(0)}
    gathers = {0: issue_gather(0)}
    stores = {}

    pes[0].wait()
    for t in range(n_tasks):
        if t + 1 < n_tasks:
            if t - 2 >= 0:
                stores[t - 2].wait()
            gathers[t + 1] = issue_gather(t + 1)
        if t == 2:
            pes[1] = issue_pe(1)
        if t == batch:
            pes[1].wait()
        gathers[t].wait()
        compute(t)
        stores[t] = issue_store(t)
    for t in range(n_tasks - _RING, n_tasks):
        stores[t].wait()


def kernel(input_ids, word_embeddings, pe):
    batch, seq_len = input_ids.shape
    ids32 = input_ids.astype(jnp.int32)
    pos_per_w = seq_len // _NW

    mesh = plsc.VectorSubcoreMesh(
        core_axis_name="c", subcore_axis_name="s",
        num_cores=_NC, num_subcores=_NS,
    )
    run = pl.kernel(
        _body,
        out_type=jax.ShapeDtypeStruct((batch, seq_len, _D), jnp.float32),
        mesh=mesh,
        scratch_types=[
            pltpu.VMEM((batch, pos_per_w), jnp.int32),
            pltpu.VMEM((_K, _D), jnp.float32),
            pltpu.VMEM((_K, _D), jnp.float32),
            pltpu.VMEM((_K, _D), jnp.float32),
            pltpu.VMEM((_K, _D), jnp.float32),
            pltpu.VMEM((_K, _D), jnp.float32),
            pltpu.SemaphoreType.DMA,
            pltpu.SemaphoreType.DMA,
            pltpu.SemaphoreType.DMA,
            pltpu.SemaphoreType.DMA,
            pltpu.SemaphoreType.DMA,
            pltpu.SemaphoreType.DMA,
            pltpu.SemaphoreType.DMA,
            pltpu.SemaphoreType.DMA,
        ],
    )
    return run(ids32, word_embeddings, pe)
